# baseline (speedup 1.0000x reference)
import functools

import jax
import jax.numpy as jnp
from jax import lax
from jax.experimental import pallas as pl
from jax.experimental.pallas import tpu as pltpu
from jax.experimental.pallas import tpu_sc as plsc

BATCH = 4096
SEQ = 50
NUM_CLASSES = 100
NUM_CORES = 2
NUM_SUBCORES = 16
NW = NUM_CORES * NUM_SUBCORES
TABLE_PAD = 128

S_TC = 32
S_SC = SEQ - S_TC
SC_CH_S = 6
SC_BW = BATCH // NW

TC_SB = 16
TC_BB = 512
TC_GS = S_TC // TC_SB
TC_GB = BATCH // TC_BB

_mesh = plsc.VectorSubcoreMesh(core_axis_name="c", subcore_axis_name="s")


@functools.partial(
    pl.kernel,
    out_type=jax.ShapeDtypeStruct((S_SC, BATCH), jnp.int32),
    mesh=_mesh,
    scratch_types=[
        pltpu.VMEM((S_SC, SC_BW), jnp.int32),
        pltpu.VMEM((TABLE_PAD,), jnp.int32),
    ],
    compiler_params=pltpu.CompilerParams(
        needs_layout_passes=False, use_tc_tiling_on_sc=True
    ),
)
def _sc_part(onehot_t_hbm, table_hbm, out_hbm, outbuf, table_v):
    wid = lax.axis_index("s") * NUM_CORES + lax.axis_index("c")
    b0 = wid * SC_BW
    pltpu.sync_copy(table_hbm, table_v)

    def run(inbuf):
        for ch in range(S_SC // SC_CH_S):
            pltpu.sync_copy(
                onehot_t_hbm.at[
                    pl.ds(S_TC + ch * SC_CH_S, SC_CH_S), :, pl.ds(b0, SC_BW)
                ],
                inbuf,
            )

            def bl_body(bl, carry, ch=ch):
                bsl = pl.ds(bl * 16, 16)
                for s_l in range(SC_CH_S):
                    accs = [jnp.zeros((16,), jnp.float32) for _ in range(4)]
                    for c in range(1, NUM_CLASSES):
                        v = inbuf[s_l, c, bsl]
                        accs[c % 4] = accs[c % 4] + v * float(c)
                    acc = (accs[0] + accs[1]) + (accs[2] + accs[3])
                    names = plsc.load_gather(table_v, [acc.astype(jnp.int32)])
                    outbuf[ch * SC_CH_S + s_l, bsl] = names
                return carry

            lax.fori_loop(0, SC_BW // 16, bl_body, 0)

    pl.run_scoped(run, pltpu.VMEM((SC_CH_S, NUM_CLASSES, SC_BW), jnp.float32))
    pltpu.sync_copy(outbuf, out_hbm.at[:, pl.ds(b0, SC_BW)])


def _tc_body(x_ref, w_ref, o_ref):
    w = w_ref[...]
    for si in range(TC_SB):
        z = lax.dot_general(
            w, x_ref[si], (((1,), (0,)), ((), ())),
            preferred_element_type=jnp.float32,
        )
        o_ref[si, :] = z.reshape(TC_BB).astype(jnp.int32)


_tc_part = pl.pallas_call(
    _tc_body,
    grid=(TC_GS, TC_GB),
    in_specs=[
        pl.BlockSpec((TC_SB, NUM_CLASSES, TC_BB), lambda i, j: (i, 0, j)),
        pl.BlockSpec((1, NUM_CLASSES), lambda i, j: (0, 0)),
    ],
    out_specs=pl.BlockSpec((TC_SB, TC_BB), lambda i, j: (i, j)),
    out_shape=jax.ShapeDtypeStruct((S_TC, BATCH), jnp.int32),
)


def kernel(onehot, idx_to_name):
    table_i = jnp.zeros((TABLE_PAD,), jnp.int32).at[:NUM_CLASSES].set(idx_to_name)
    wtile = idx_to_name.astype(jnp.float32).reshape(1, NUM_CLASSES)
    onehot_t = jnp.transpose(onehot, (1, 2, 0))
    sc_out = _sc_part(onehot_t, table_i)
    tc_out = _tc_part(onehot_t, wtile)
    out_t = jnp.concatenate([tc_out, sc_out], axis=0)
    return jnp.transpose(out_t, (1, 0))


# speedup vs baseline: 1.0025x; 1.0025x over previous
import functools

import jax
import jax.numpy as jnp
from jax import lax
from jax.experimental import pallas as pl
from jax.experimental.pallas import tpu as pltpu
from jax.experimental.pallas import tpu_sc as plsc

BATCH = 4096
SEQ = 50
NUM_CLASSES = 100
NUM_CORES = 2
NUM_SUBCORES = 16
NW = NUM_CORES * NUM_SUBCORES
TABLE_PAD = 128

S_TC = 32
S_SC = SEQ - S_TC
SC_CH_S = 6
SC_BW = BATCH // NW

TC_SB = 8
TC_BB = 1024
TC_GS = S_TC // TC_SB
TC_GB = BATCH // TC_BB

_mesh = plsc.VectorSubcoreMesh(core_axis_name="c", subcore_axis_name="s")


@functools.partial(
    pl.kernel,
    out_type=jax.ShapeDtypeStruct((S_SC, BATCH), jnp.int32),
    mesh=_mesh,
    scratch_types=[
        pltpu.VMEM((S_SC, SC_BW), jnp.int32),
        pltpu.VMEM((TABLE_PAD,), jnp.int32),
    ],
    compiler_params=pltpu.CompilerParams(
        needs_layout_passes=False, use_tc_tiling_on_sc=True
    ),
)
def _sc_part(onehot_t_hbm, table_hbm, out_hbm, outbuf, table_v):
    wid = lax.axis_index("s") * NUM_CORES + lax.axis_index("c")
    b0 = wid * SC_BW
    pltpu.sync_copy(table_hbm, table_v)

    def run(inbuf):
        for ch in range(S_SC // SC_CH_S):
            pltpu.sync_copy(
                onehot_t_hbm.at[
                    pl.ds(S_TC + ch * SC_CH_S, SC_CH_S), :, pl.ds(b0, SC_BW)
                ],
                inbuf,
            )

            def bl_body(bl, carry, ch=ch):
                bsl = pl.ds(bl * 16, 16)
                for s_l in range(SC_CH_S):
                    accs = [jnp.zeros((16,), jnp.float32) for _ in range(4)]
                    for c in range(1, NUM_CLASSES):
                        v = inbuf[s_l, c, bsl]
                        accs[c % 4] = accs[c % 4] + v * float(c)
                    acc = (accs[0] + accs[1]) + (accs[2] + accs[3])
                    names = plsc.load_gather(table_v, [acc.astype(jnp.int32)])
                    outbuf[ch * SC_CH_S + s_l, bsl] = names
                return carry

            lax.fori_loop(0, SC_BW // 16, bl_body, 0)

    pl.run_scoped(run, pltpu.VMEM((SC_CH_S, NUM_CLASSES, SC_BW), jnp.float32))
    pltpu.sync_copy(outbuf, out_hbm.at[:, pl.ds(b0, SC_BW)])


def _tc_body(x_ref, w_ref, o_ref):
    w = w_ref[...]
    for si in range(TC_SB):
        z = lax.dot_general(
            w, x_ref[si], (((1,), (0,)), ((), ())),
            preferred_element_type=jnp.float32,
        )
        o_ref[si, :] = z.reshape(TC_BB).astype(jnp.int32)


_tc_part = pl.pallas_call(
    _tc_body,
    grid=(TC_GS, TC_GB),
    in_specs=[
        pl.BlockSpec((TC_SB, NUM_CLASSES, TC_BB), lambda i, j: (i, 0, j)),
        pl.BlockSpec((1, NUM_CLASSES), lambda i, j: (0, 0)),
    ],
    out_specs=pl.BlockSpec((TC_SB, TC_BB), lambda i, j: (i, j)),
    out_shape=jax.ShapeDtypeStruct((S_TC, BATCH), jnp.int32),
)


def kernel(onehot, idx_to_name):
    table_i = jnp.zeros((TABLE_PAD,), jnp.int32).at[:NUM_CLASSES].set(idx_to_name)
    wtile = idx_to_name.astype(jnp.float32).reshape(1, NUM_CLASSES)
    onehot_t = jnp.transpose(onehot, (1, 2, 0))
    sc_out = _sc_part(onehot_t, table_i)
    tc_out = _tc_part(onehot_t, wtile)
    out_t = jnp.concatenate([tc_out, sc_out], axis=0)
    return jnp.transpose(out_t, (1, 0))


# speedup vs baseline: 1.0201x; 1.0175x over previous
import functools

import jax
import jax.numpy as jnp
from jax import lax
from jax.experimental import pallas as pl
from jax.experimental.pallas import tpu as pltpu
from jax.experimental.pallas import tpu_sc as plsc

BATCH = 4096
SEQ = 50
NUM_CLASSES = 100
NUM_CORES = 2
NUM_SUBCORES = 16
NW = NUM_CORES * NUM_SUBCORES
TABLE_PAD = 128

S_TC = 32
S_SC = SEQ - S_TC
SC_CH_S = 6
SC_BW = BATCH // NW

TC_SB = 8
TC_BB = 512
TC_GS = S_TC // TC_SB
TC_GB = BATCH // TC_BB

_mesh = plsc.VectorSubcoreMesh(core_axis_name="c", subcore_axis_name="s")


@functools.partial(
    pl.kernel,
    out_type=jax.ShapeDtypeStruct((S_SC, BATCH), jnp.int32),
    mesh=_mesh,
    scratch_types=[
        pltpu.VMEM((S_SC, SC_BW), jnp.int32),
        pltpu.VMEM((TABLE_PAD,), jnp.int32),
    ],
    compiler_params=pltpu.CompilerParams(
        needs_layout_passes=False, use_tc_tiling_on_sc=True
    ),
)
def _sc_part(onehot_t_hbm, table_hbm, out_hbm, outbuf, table_v):
    wid = lax.axis_index("s") * NUM_CORES + lax.axis_index("c")
    b0 = wid * SC_BW
    pltpu.sync_copy(table_hbm, table_v)

    def run(inbuf):
        for ch in range(S_SC // SC_CH_S):
            pltpu.sync_copy(
                onehot_t_hbm.at[
                    pl.ds(S_TC + ch * SC_CH_S, SC_CH_S), :, pl.ds(b0, SC_BW)
                ],
                inbuf,
            )

            def bl_body(bl, carry, ch=ch):
                bsl = pl.ds(bl * 16, 16)
                for s_l in range(SC_CH_S):
                    accs = [jnp.zeros((16,), jnp.float32) for _ in range(4)]
                    for c in range(1, NUM_CLASSES):
                        v = inbuf[s_l, c, bsl]
                        accs[c % 4] = accs[c % 4] + v * float(c)
                    acc = (accs[0] + accs[1]) + (accs[2] + accs[3])
                    names = plsc.load_gather(table_v, [acc.astype(jnp.int32)])
                    outbuf[ch * SC_CH_S + s_l, bsl] = names
                return carry

            lax.fori_loop(0, SC_BW // 16, bl_body, 0)

    pl.run_scoped(run, pltpu.VMEM((SC_CH_S, NUM_CLASSES, SC_BW), jnp.float32))
    pltpu.sync_copy(outbuf, out_hbm.at[:, pl.ds(b0, SC_BW)])


def _tc_body(x_ref, w_ref, o_ref):
    w = w_ref[...]
    for si in range(TC_SB):
        z = lax.dot_general(
            w, x_ref[si], (((1,), (0,)), ((), ())),
            preferred_element_type=jnp.float32,
        )
        o_ref[si, :] = z.reshape(TC_BB).astype(jnp.int32)


_tc_part = pl.pallas_call(
    _tc_body,
    grid=(TC_GS, TC_GB),
    in_specs=[
        pl.BlockSpec((TC_SB, NUM_CLASSES, TC_BB), lambda i, j: (i, 0, j)),
        pl.BlockSpec((1, NUM_CLASSES), lambda i, j: (0, 0)),
    ],
    out_specs=pl.BlockSpec((TC_SB, TC_BB), lambda i, j: (i, j)),
    out_shape=jax.ShapeDtypeStruct((S_TC, BATCH), jnp.int32),
)


def kernel(onehot, idx_to_name):
    table_i = jnp.zeros((TABLE_PAD,), jnp.int32).at[:NUM_CLASSES].set(idx_to_name)
    wtile = idx_to_name.astype(jnp.float32).reshape(1, NUM_CLASSES)
    onehot_t = jnp.transpose(onehot, (1, 2, 0))
    sc_out = _sc_part(onehot_t, table_i)
    tc_out = _tc_part(onehot_t, wtile)
    out_t = jnp.concatenate([tc_out, sc_out], axis=0)
    return jnp.transpose(out_t, (1, 0))
